# Initial kernel scaffold; baseline (speedup 1.0000x reference)
import jax
import jax.numpy as jnp
from jax.experimental import pallas as pl


def kernel(x, edge_index, W1, b1, W2, b2):
    raise NotImplementedError("write your pallas kernel here")


# speedup vs baseline: 22.7604x; 22.7604x over previous
import functools

import jax
import jax.numpy as jnp
from jax import lax
from jax.experimental import pallas as pl
from jax.experimental.pallas import tpu as pltpu
from jax.experimental.pallas import tpu_sc as plsc

N = 10000
NPAD = 10240
E = 160000
NW = 32
CH = 128
K = 40
EPAD = NW * K * CH
DUMMY = 10016
ROWS_PER_TILE = NPAD // 16

_mesh = plsc.VectorSubcoreMesh(core_axis_name="c", subcore_axis_name="s")
_sc_params = pltpu.CompilerParams(use_tc_tiling_on_sc=False)


@functools.partial(
    pl.kernel,
    out_type=jax.ShapeDtypeStruct((2, NPAD), jnp.float32),
    mesh=_mesh,
    compiler_params=_sc_params,
    scratch_types=[
        pltpu.VMEM((K, CH), jnp.int32),
        pltpu.VMEM((CH,), jnp.float32),
        pltpu.VMEM_SHARED((NPAD,), jnp.float32),
    ],
)
def _deg_kernel(dst_hbm, ones_hbm, zeros_hbm, out_hbm, idx_v, ones_v, deg_sh):
    c = lax.axis_index("c")
    s = lax.axis_index("s")
    wid = c * 16 + s
    @pl.when(s == 0)
    def _():
        pltpu.sync_copy(zeros_hbm, deg_sh)
    pltpu.sync_copy(ones_hbm, ones_v)
    pltpu.sync_copy(dst_hbm.at[wid], idx_v)
    plsc.subcore_barrier()

    def body(j, _):
        pltpu.sync_copy(ones_v, deg_sh.at[idx_v.at[j]], add=True)
        return 0

    lax.fori_loop(0, K, body, 0)
    plsc.subcore_barrier()
    sl = pl.ds(s * ROWS_PER_TILE, ROWS_PER_TILE)
    pltpu.sync_copy(deg_sh.at[sl], out_hbm.at[c, sl])


def _make_agg(width):
    @functools.partial(
        pl.kernel,
        out_type=jax.ShapeDtypeStruct((2, NPAD, width), jnp.float32),
        mesh=_mesh,
        compiler_params=_sc_params,
        scratch_types=[
            pltpu.VMEM((K, CH), jnp.int32),
            pltpu.VMEM((K, CH), jnp.int32),
            pltpu.VMEM((CH, width), jnp.float32),
            pltpu.VMEM_SHARED((NPAD, width), jnp.float32),
            pltpu.SemaphoreType.DMA,
        ],
    )
    def agg(src_hbm, dst_hbm, feat_hbm, zeros_hbm, out_hbm,
            src_v, dst_v, rows_v, acc_sh, sem):
        c = lax.axis_index("c")
        s = lax.axis_index("s")
        wid = c * 16 + s
        @pl.when(s == 0)
        def _():
            pltpu.sync_copy(zeros_hbm, acc_sh)
        pltpu.sync_copy(src_hbm.at[wid], src_v)
        pltpu.sync_copy(dst_hbm.at[wid], dst_v)
        plsc.subcore_barrier()

        def body(j, _):
            pltpu.async_copy(feat_hbm.at[src_v.at[j]], rows_v, sem).wait()
            pltpu.sync_copy(rows_v, acc_sh.at[dst_v.at[j]], add=True)
            return 0

        lax.fori_loop(0, K, body, 0)
        plsc.subcore_barrier()
        sl = pl.ds(s * ROWS_PER_TILE, ROWS_PER_TILE)
        pltpu.sync_copy(acc_sh.at[sl], out_hbm.at[c, sl])

    return agg


_agg16 = _make_agg(16)


def _tc1_body(x_ref, w1_ref, dp0_ref, dp1_ref, xs_ref, dinv_ref):
    deg = dp0_ref[...] + dp1_ref[...] + 1.0
    dinv = lax.rsqrt(deg)
    dinv_ref[...] = dinv
    xw = jnp.dot(x_ref[...], w1_ref[...], preferred_element_type=jnp.float32)
    xs_ref[0:N, :] = xw * dinv[0:N]
    xs_ref[N:NPAD, :] = jnp.zeros((NPAD - N, 16), jnp.float32)


def _tc2_body(a0_ref, a1_ref, xs_ref, dinv_ref, b1_ref, hs_ref):
    dinv = dinv_ref[...]
    h = jnp.maximum(dinv * (a0_ref[...] + a1_ref[...] + xs_ref[...])
                    + b1_ref[...], 0.0)
    hs = h * dinv
    hs_ref[0:N, :] = hs[0:N]
    hs_ref[N:NPAD, :] = jnp.zeros((NPAD - N, 16), jnp.float32)


def _tc3_body(a0_ref, a1_ref, hs_ref, dinv_ref, w2_ref, b2_ref, out_ref):
    m = dinv_ref[0:N] * (a0_ref[0:N] + a1_ref[0:N] + hs_ref[0:N])
    out_ref[...] = (jnp.dot(m, w2_ref[...],
                            preferred_element_type=jnp.float32)
                    + b2_ref[...])


def kernel(x, edge_index, W1, b1, W2, b2):
    f32 = jnp.float32
    src = edge_index[0].astype(jnp.int32)
    dst = edge_index[1].astype(jnp.int32)
    pad = DUMMY + (jnp.arange(EPAD - E, dtype=jnp.int32) % 128)
    src3 = jnp.concatenate([src, pad]).reshape(NW, K, CH)
    dst3 = jnp.concatenate([dst, pad]).reshape(NW, K, CH)

    ones_ch = jnp.ones((CH,), f32)
    z1 = jnp.zeros((NPAD,), f32)
    z16 = jnp.zeros((NPAD, 16), f32)

    degp = _deg_kernel(dst3, ones_ch, z1)

    dp0 = degp[0].reshape(NPAD, 1)
    dp1 = degp[1].reshape(NPAD, 1)
    xs, dinv = pl.pallas_call(
        _tc1_body,
        out_shape=(jax.ShapeDtypeStruct((NPAD, 16), f32),
                   jax.ShapeDtypeStruct((NPAD, 1), f32)),
    )(x, W1, dp0, dp1)

    accp = _agg16(src3, dst3, xs, z16)

    hs = pl.pallas_call(
        _tc2_body,
        out_shape=jax.ShapeDtypeStruct((NPAD, 16), f32),
    )(accp[0], accp[1], xs, dinv, b1)

    accp2 = _agg16(src3, dst3, hs, z16)

    out = pl.pallas_call(
        _tc3_body,
        out_shape=jax.ShapeDtypeStruct((N, 2), f32),
    )(accp2[0], accp2[1], hs, dinv, W2, b2)
    return out


# speedup vs baseline: 51.8485x; 2.2780x over previous
import functools

import jax
import jax.numpy as jnp
from jax import lax
from jax.experimental import pallas as pl
from jax.experimental.pallas import tpu as pltpu
from jax.experimental.pallas import tpu_sc as plsc

N = 10000
NPAD = 10240
E = 160000
NW = 32
CH = 125
K = 40
EPT = K * CH
RPT = NPAD // 16

_mesh = plsc.VectorSubcoreMesh(core_axis_name="c", subcore_axis_name="s")
_sc_params = pltpu.CompilerParams(use_tc_tiling_on_sc=False)


@functools.partial(
    pl.kernel,
    out_type=jax.ShapeDtypeStruct((2, NPAD), jnp.float32),
    mesh=_mesh,
    compiler_params=_sc_params,
    scratch_types=[
        pltpu.VMEM((K, CH), jnp.int32),
        pltpu.VMEM((128,), jnp.float32),
        pltpu.VMEM((RPT,), jnp.float32),
        pltpu.VMEM_SHARED((NPAD,), jnp.float32),
        pltpu.SemaphoreType.DMA,
    ],
)
def _deg_kernel(ei_hbm, out_hbm, idx_v, ones_v, zer_v, deg_sh, sem):
    c = lax.axis_index("c")
    s = lax.axis_index("s")
    wid = c * 16 + s
    one = jnp.ones((16,), jnp.float32)
    zero = jnp.zeros((16,), jnp.float32)

    def fill_ones(i, _):
        ones_v[pl.ds(i * 16, 16)] = one
        return 0

    lax.fori_loop(0, 8, fill_ones, 0)

    def fill_zero(i, _):
        zer_v[pl.ds(i * 16, 16)] = zero
        return 0

    lax.fori_loop(0, RPT // 16, fill_zero, 0)
    pltpu.sync_copy(ei_hbm.at[1, wid], idx_v)
    pltpu.sync_copy(zer_v, deg_sh.at[pl.ds(s * RPT, RPT)])
    plsc.subcore_barrier()

    def fire(j, _):
        pltpu.async_copy(ones_v.at[pl.ds(0, CH)], deg_sh.at[idx_v.at[j]],
                         sem, add=True)
        return 0

    lax.fori_loop(0, K, fire, 0)

    def drain(j, _):
        pltpu.make_async_copy(ones_v.at[pl.ds(0, CH)],
                              deg_sh.at[idx_v.at[j]], sem).wait()
        return 0

    lax.fori_loop(0, K, drain, 0)
    plsc.subcore_barrier()
    sl = pl.ds(s * RPT, RPT)
    pltpu.sync_copy(deg_sh.at[sl], out_hbm.at[c, sl])


@functools.partial(
    pl.kernel,
    out_type=jax.ShapeDtypeStruct((2, NPAD, 16), jnp.float32),
    mesh=_mesh,
    compiler_params=_sc_params,
    scratch_types=[
        pltpu.VMEM((K, CH), jnp.int32),
        pltpu.VMEM((K, CH), jnp.int32),
        pltpu.VMEM((EPT, 16), jnp.float32),
        pltpu.VMEM((128, 16), jnp.float32),
        pltpu.VMEM_SHARED((NPAD, 16), jnp.float32),
        pltpu.SemaphoreType.DMA,
        pltpu.SemaphoreType.DMA,
    ],
)
def _agg_kernel(ei_hbm, feat_hbm, out_hbm,
                src_v, dst_v, rows_v, zer_v, acc_sh, semg, sems):
    c = lax.axis_index("c")
    s = lax.axis_index("s")
    wid = c * 16 + s
    zero = jnp.zeros((16,), jnp.float32)

    def fill_zero(i, _):
        zer_v[i] = zero
        return 0

    lax.fori_loop(0, 128, fill_zero, 0)
    pltpu.sync_copy(ei_hbm.at[0, wid], src_v)
    pltpu.sync_copy(ei_hbm.at[1, wid], dst_v)

    def zero_acc(t, _):
        pltpu.sync_copy(zer_v, acc_sh.at[pl.ds(s * RPT + t * 128, 128)])
        return 0

    lax.fori_loop(0, RPT // 128, zero_acc, 0)
    plsc.subcore_barrier()

    def fire_g(j, _):
        pltpu.async_copy(feat_hbm.at[src_v.at[j]],
                         rows_v.at[pl.ds(j * CH, CH)], semg)
        return 0

    lax.fori_loop(0, K, fire_g, 0)
    pltpu.make_async_copy(feat_hbm.at[pl.ds(0, EPT)], rows_v, semg).wait()

    def fire_s(j, _):
        pltpu.async_copy(rows_v.at[pl.ds(j * CH, CH)],
                         acc_sh.at[dst_v.at[j]], sems, add=True)
        return 0

    lax.fori_loop(0, K, fire_s, 0)
    pltpu.make_async_copy(rows_v, acc_sh.at[pl.ds(0, EPT)], sems).wait()
    plsc.subcore_barrier()
    sl = pl.ds(s * RPT, RPT)
    pltpu.sync_copy(acc_sh.at[sl], out_hbm.at[c, sl])


def _tc1_body(x_ref, w1_ref, degp_ref, xs_ref, dinvw_ref):
    dp = degp_ref[...]
    deg = dp[0, :] + dp[1, :] + 1.0
    dinv = lax.rsqrt(deg)
    dcol = dinv[:, None]
    dinvw_ref[...] = jnp.broadcast_to(dcol, (NPAD, 16))
    xw = jnp.dot(x_ref[...], w1_ref[...], preferred_element_type=jnp.float32)
    xs_ref[0:N, :] = xw * dcol[0:N]
    xs_ref[N:NPAD, :] = jnp.zeros((NPAD - N, 16), jnp.float32)


def _tc2_body(accp_ref, xs_ref, dinvw_ref, b1w_ref, hs_ref):
    a = accp_ref[0] + accp_ref[1]
    dw = dinvw_ref[...]
    h = jnp.maximum(dw * (a + xs_ref[...]) + b1w_ref[...], 0.0)
    hs_ref[...] = h * dw


def _tc3_body(accp2_ref, hs_ref, dinvw_ref, w2big_ref, b2w_ref, out_ref):
    m = dinvw_ref[...] * (accp2_ref[0] + accp2_ref[1] + hs_ref[...])
    out_ref[...] = (jnp.dot(m, w2big_ref[...],
                            preferred_element_type=jnp.float32)
                    + b2w_ref[...])


def kernel(x, edge_index, W1, b1, W2, b2):
    f32 = jnp.float32
    ei4 = edge_index.astype(jnp.int32).reshape(2, NW, K, CH)

    degp = _deg_kernel(ei4)

    xs, dinvw = pl.pallas_call(
        _tc1_body,
        out_shape=(jax.ShapeDtypeStruct((NPAD, 16), f32),
                   jax.ShapeDtypeStruct((NPAD, 16), f32)),
    )(x, W1, degp)

    accp = _agg_kernel(ei4, xs)

    accp_w = accp.reshape(2, NPAD // 8, 128)
    xs_w = xs.reshape(NPAD // 8, 128)
    dinvw_w = dinvw.reshape(NPAD // 8, 128)
    b1w = jnp.tile(b1, 8)
    hs_w = pl.pallas_call(
        _tc2_body,
        out_shape=jax.ShapeDtypeStruct((NPAD // 8, 128), f32),
    )(accp_w, xs_w, dinvw_w, b1w)
    hs = hs_w.reshape(NPAD, 16)

    accp2 = _agg_kernel(ei4, hs)

    w2big = jnp.kron(jnp.eye(8, dtype=f32), W2.astype(f32))
    b2w = jnp.tile(b2, 8)
    out_w = pl.pallas_call(
        _tc3_body,
        out_shape=jax.ShapeDtypeStruct((NPAD // 8, 16), f32),
    )(accp2.reshape(2, NPAD // 8, 128), hs_w, dinvw_w, w2big, b2w)
    return out_w.reshape(NPAD, 2)[:N]


# speedup vs baseline: 54.0396x; 1.0423x over previous
import functools

import jax
import jax.numpy as jnp
from jax import lax
from jax.experimental import pallas as pl
from jax.experimental.pallas import tpu as pltpu
from jax.experimental.pallas import tpu_sc as plsc

N = 10000
NPAD = 10240
NW8 = NPAD // 8
E = 160000
NW = 32
EPT = E // NW
CH = 128
K = EPT // CH
TAIL = EPT - K * CH
EPT_D = E // 16
K_D = EPT_D // CH
TAIL_D = EPT_D - K_D * CH
RPT = NPAD // 16
RPW = NPAD // 32

_mesh = plsc.VectorSubcoreMesh(core_axis_name="c", subcore_axis_name="s")
_sc_params = pltpu.CompilerParams(use_tc_tiling_on_sc=False,
                                  needs_layout_passes=False)


@functools.partial(
    pl.kernel,
    out_type=(jax.ShapeDtypeStruct((NPAD,), jnp.float32),
              jax.ShapeDtypeStruct((NPAD, 16), jnp.float32)),
    mesh=_mesh,
    compiler_params=_sc_params,
    scratch_types=[
        pltpu.VMEM((EPT_D,), jnp.int32),
        pltpu.VMEM((128,), jnp.float32),
        pltpu.VMEM((RPT,), jnp.float32),
        pltpu.VMEM((RPW,), jnp.float32),
        pltpu.VMEM((RPW, 16), jnp.float32),
        pltpu.VMEM_SHARED((NPAD,), jnp.float32),
        pltpu.SemaphoreType.DMA,
    ],
)
def _deg_kernel(ei_hbm, dinv_hbm, dinvw_hbm,
                idx_v, ones_v, zer_v, dinv_v, dvw_v, deg_sh, sem):
    c = lax.axis_index("c")
    s = lax.axis_index("s")
    wid = c * 16 + s
    one = jnp.ones((16,), jnp.float32)
    zero = jnp.zeros((16,), jnp.float32)

    def fill_ones(i, _):
        ones_v[pl.ds(i * 16, 16)] = one
        return 0

    lax.fori_loop(0, 8, fill_ones, 0)

    def fill_zero(i, _):
        zer_v[pl.ds(i * 16, 16)] = zero
        return 0

    lax.fori_loop(0, RPT // 16, fill_zero, 0)
    pltpu.sync_copy(ei_hbm.at[pl.ds(E + s * EPT_D, EPT_D)], idx_v)
    pltpu.sync_copy(zer_v, deg_sh.at[pl.ds(s * RPT, RPT)])
    plsc.subcore_barrier()

    def fire(j, _):
        pltpu.async_copy(ones_v.at[pl.ds(0, CH)],
                         deg_sh.at[idx_v.at[pl.ds(j * CH, CH)]], sem, add=True)
        return 0

    lax.fori_loop(0, K_D, fire, 0)
    pltpu.async_copy(ones_v.at[pl.ds(0, TAIL_D)],
                     deg_sh.at[idx_v.at[pl.ds(K_D * CH, TAIL_D)]], sem,
                     add=True)

    def drain(j, _):
        pltpu.make_async_copy(ones_v.at[pl.ds(0, CH)],
                              deg_sh.at[idx_v.at[pl.ds(j * CH, CH)]],
                              sem).wait()
        return 0

    lax.fori_loop(0, K_D, drain, 0)
    pltpu.make_async_copy(ones_v.at[pl.ds(0, TAIL_D)],
                          deg_sh.at[idx_v.at[pl.ds(K_D * CH, TAIL_D)]],
                          sem).wait()
    plsc.subcore_barrier()

    pltpu.sync_copy(deg_sh.at[pl.ds(wid * RPW, RPW)], dinv_v)

    def rsqrt_chunk(i, _):
        d = dinv_v[pl.ds(i * 16, 16)] + 1.0
        bits = plsc.bitcast(d, jnp.int32)
        y = plsc.bitcast(0x5F3759DF - lax.shift_right_logical(bits, 1),
                         jnp.float32)
        half = -0.5 * d
        y = y * (1.5 + half * y * y)
        y = y * (1.5 + half * y * y)
        y = y * (1.5 + half * y * y)
        dinv_v[pl.ds(i * 16, 16)] = y
        return 0

    lax.fori_loop(0, RPW // 16, rsqrt_chunk, 0)

    def splat_row(r, _):
        dvw_v[r] = plsc.load_gather(dinv_v, [jnp.full((16,), r, jnp.int32)])
        return 0

    lax.fori_loop(0, RPW, splat_row, 0)
    sl = pl.ds(wid * RPW, RPW)
    pltpu.sync_copy(dinv_v, dinv_hbm.at[sl])
    pltpu.sync_copy(dvw_v, dinvw_hbm.at[sl])


@functools.partial(
    pl.kernel,
    out_type=jax.ShapeDtypeStruct((2, NPAD, 16), jnp.float32),
    mesh=_mesh,
    compiler_params=_sc_params,
    scratch_types=[
        pltpu.VMEM((EPT,), jnp.int32),
        pltpu.VMEM((EPT,), jnp.int32),
        pltpu.VMEM((EPT, 16), jnp.float32),
        pltpu.VMEM((128, 16), jnp.float32),
        pltpu.VMEM_SHARED((NPAD, 16), jnp.float32),
        pltpu.SemaphoreType.DMA,
        pltpu.SemaphoreType.DMA,
    ],
)
def _agg_kernel(ei_hbm, feat_hbm, out_hbm,
                src_v, dst_v, rows_v, zer_v, acc_sh, semg, sems):
    c = lax.axis_index("c")
    s = lax.axis_index("s")
    wid = c * 16 + s
    zero = jnp.zeros((16,), jnp.float32)

    def fill_zero(i, _):
        zer_v[i] = zero
        return 0

    lax.fori_loop(0, 128, fill_zero, 0)
    pltpu.sync_copy(ei_hbm.at[pl.ds(wid * EPT, EPT)], src_v)
    pltpu.sync_copy(ei_hbm.at[pl.ds(E + wid * EPT, EPT)], dst_v)

    def zero_acc(t, _):
        pltpu.sync_copy(zer_v, acc_sh.at[pl.ds(s * RPT + t * 128, 128)])
        return 0

    lax.fori_loop(0, RPT // 128, zero_acc, 0)
    plsc.subcore_barrier()

    def fire_g(j, _):
        pltpu.async_copy(feat_hbm.at[src_v.at[pl.ds(j * CH, CH)]],
                         rows_v.at[pl.ds(j * CH, CH)], semg)
        return 0

    lax.fori_loop(0, K, fire_g, 0)
    pltpu.async_copy(feat_hbm.at[src_v.at[pl.ds(K * CH, TAIL)]],
                     rows_v.at[pl.ds(K * CH, TAIL)], semg)
    pltpu.make_async_copy(feat_hbm.at[pl.ds(0, EPT)], rows_v, semg).wait()

    def fire_s(j, _):
        pltpu.async_copy(rows_v.at[pl.ds(j * CH, CH)],
                         acc_sh.at[dst_v.at[pl.ds(j * CH, CH)]], sems, add=True)
        return 0

    lax.fori_loop(0, K, fire_s, 0)
    pltpu.async_copy(rows_v.at[pl.ds(K * CH, TAIL)],
                     acc_sh.at[dst_v.at[pl.ds(K * CH, TAIL)]], sems, add=True)
    pltpu.make_async_copy(rows_v, acc_sh.at[pl.ds(0, EPT)], sems).wait()
    plsc.subcore_barrier()
    sl = pl.ds(s * RPT, RPT)
    pltpu.sync_copy(acc_sh.at[sl], out_hbm.at[c, sl])


def _tc1_body(x_ref, w1_ref, dinv_ref, xs_ref):
    dcol = dinv_ref[...][:, None]
    xw = jnp.dot(x_ref[...], w1_ref[...], preferred_element_type=jnp.float32)
    xs_ref[0:N, :] = xw * dcol[0:N]
    xs_ref[N:NPAD, :] = jnp.zeros((NPAD - N, 16), jnp.float32)


def _tc2_body(accp_ref, xs_ref, dinvw_ref, b1_ref, hs_ref):
    a = accp_ref[0] + accp_ref[1]
    dw = dinvw_ref[...]
    b1w = jnp.concatenate([b1_ref[...]] * 8)
    h = jnp.maximum(dw * (a + xs_ref[...]) + b1w, 0.0)
    hs_ref[...] = h * dw


def _tc3_body(accp2_ref, hs_ref, dinvw_ref, w2big_ref, b2_ref, out_ref):
    m = dinvw_ref[...] * (accp2_ref[0] + accp2_ref[1] + hs_ref[...])
    b2w = jnp.concatenate([b2_ref[...]] * 8)
    out_ref[...] = (jnp.dot(m, w2big_ref[...],
                            preferred_element_type=jnp.float32) + b2w)


def kernel(x, edge_index, W1, b1, W2, b2):
    f32 = jnp.float32
    ei_lin = edge_index.astype(jnp.int32).reshape(2 * E)

    dinv, dinvw = _deg_kernel(ei_lin)

    xs = pl.pallas_call(
        _tc1_body,
        out_shape=jax.ShapeDtypeStruct((NPAD, 16), f32),
    )(x, W1, dinv)

    accp = _agg_kernel(ei_lin, xs)

    dinvw_w = dinvw.reshape(NW8, 128)
    hs_w = pl.pallas_call(
        _tc2_body,
        out_shape=jax.ShapeDtypeStruct((NW8, 128), f32),
    )(accp.reshape(2, NW8, 128), xs.reshape(NW8, 128), dinvw_w, b1)

    accp2 = _agg_kernel(ei_lin, hs_w.reshape(NPAD, 16))

    w2big = jnp.kron(jnp.eye(8, dtype=f32), W2.astype(f32))
    out_w = pl.pallas_call(
        _tc3_body,
        out_shape=jax.ShapeDtypeStruct((NW8, 16), f32),
    )(accp2.reshape(2, NW8, 128), hs_w, dinvw_w, w2big, b2)
    return out_w[:N * 2 // 16].reshape(N, 2)


# speedup vs baseline: 56.9940x; 1.0547x over previous
import functools

import jax
import jax.numpy as jnp
from jax import lax
from jax.experimental import pallas as pl
from jax.experimental.pallas import tpu as pltpu
from jax.experimental.pallas import tpu_sc as plsc

N = 10000
NPAD = 10240
NW8 = NPAD // 8
E = 160000
NW = 32
EPT = E // NW
CH = 128
K = EPT // CH
TAIL = EPT - K * CH
EPT_D = E // 16
K_D = EPT_D // CH
TAIL_D = EPT_D - K_D * CH
RPT = NPAD // 16
RPW = NPAD // 32

_mesh = plsc.VectorSubcoreMesh(core_axis_name="c", subcore_axis_name="s")
_sc_params = pltpu.CompilerParams(use_tc_tiling_on_sc=False,
                                  needs_layout_passes=False)


@functools.partial(
    pl.kernel,
    out_type=(jax.ShapeDtypeStruct((NPAD,), jnp.float32),
              jax.ShapeDtypeStruct((NPAD, 16), jnp.float32)),
    mesh=_mesh,
    compiler_params=_sc_params,
    scratch_types=[
        pltpu.VMEM((EPT_D,), jnp.int32),
        pltpu.VMEM((128,), jnp.float32),
        pltpu.VMEM((RPT,), jnp.float32),
        pltpu.VMEM((RPW,), jnp.float32),
        pltpu.VMEM((RPW, 16), jnp.float32),
        pltpu.VMEM_SHARED((NPAD,), jnp.float32),
        pltpu.SemaphoreType.DMA,
    ],
)
def _deg_kernel(ei_hbm, dinv_hbm, dinvw_hbm,
                idx_v, ones_v, zer_v, dinv_v, dvw_v, deg_sh, sem):
    c = lax.axis_index("c")
    s = lax.axis_index("s")
    wid = c * 16 + s
    one = jnp.ones((16,), jnp.float32)
    zero = jnp.zeros((16,), jnp.float32)

    def fill_ones(i, _):
        ones_v[pl.ds(i * 16, 16)] = one
        return 0

    lax.fori_loop(0, 8, fill_ones, 0)

    def fill_zero(i, _):
        zer_v[pl.ds(i * 16, 16)] = zero
        return 0

    lax.fori_loop(0, RPT // 16, fill_zero, 0)
    pltpu.sync_copy(ei_hbm.at[pl.ds(E + s * EPT_D, EPT_D)], idx_v)
    pltpu.sync_copy(zer_v, deg_sh.at[pl.ds(s * RPT, RPT)])
    plsc.subcore_barrier()

    def fire(j, _):
        pltpu.async_copy(ones_v.at[pl.ds(0, CH)],
                         deg_sh.at[idx_v.at[pl.ds(j * CH, CH)]], sem, add=True)
        return 0

    lax.fori_loop(0, K_D, fire, 0)
    pltpu.async_copy(ones_v.at[pl.ds(0, TAIL_D)],
                     deg_sh.at[idx_v.at[pl.ds(K_D * CH, TAIL_D)]], sem,
                     add=True)

    def drain(j, _):
        pltpu.make_async_copy(ones_v.at[pl.ds(0, CH)],
                              deg_sh.at[idx_v.at[pl.ds(j * CH, CH)]],
                              sem).wait()
        return 0

    lax.fori_loop(0, K_D, drain, 0)
    pltpu.make_async_copy(ones_v.at[pl.ds(0, TAIL_D)],
                          deg_sh.at[idx_v.at[pl.ds(K_D * CH, TAIL_D)]],
                          sem).wait()
    plsc.subcore_barrier()

    pltpu.sync_copy(deg_sh.at[pl.ds(wid * RPW, RPW)], dinv_v)

    def rsqrt_chunk(i, _):
        d = dinv_v[pl.ds(i * 16, 16)] + 1.0
        bits = plsc.bitcast(d, jnp.int32)
        y = plsc.bitcast(0x5F3759DF - lax.shift_right_logical(bits, 1),
                         jnp.float32)
        half = -0.5 * d
        y = y * (1.5 + half * y * y)
        y = y * (1.5 + half * y * y)
        y = y * (1.5 + half * y * y)
        dinv_v[pl.ds(i * 16, 16)] = y
        return 0

    lax.fori_loop(0, RPW // 16, rsqrt_chunk, 0)

    def splat_row(r, _):
        dvw_v[r] = plsc.load_gather(dinv_v, [jnp.full((16,), r, jnp.int32)])
        return 0

    lax.fori_loop(0, RPW, splat_row, 0)
    sl = pl.ds(wid * RPW, RPW)
    pltpu.sync_copy(dinv_v, dinv_hbm.at[sl])
    pltpu.sync_copy(dvw_v, dinvw_hbm.at[sl])


@functools.partial(
    pl.kernel,
    out_type=jax.ShapeDtypeStruct((2, NPAD, 16), jnp.float32),
    mesh=_mesh,
    compiler_params=_sc_params,
    scratch_types=[
        pltpu.VMEM((EPT,), jnp.int32),
        pltpu.VMEM((EPT,), jnp.int32),
        pltpu.VMEM((EPT, 16), jnp.float32),
        pltpu.VMEM((128, 16), jnp.float32),
        pltpu.VMEM_SHARED((NPAD, 16), jnp.float32),
        pltpu.SemaphoreType.DMA,
        pltpu.SemaphoreType.DMA,
        pltpu.SemaphoreType.DMA,
    ],
)
def _agg_kernel(ei_hbm, feat_hbm, out_hbm,
                src_v, dst_v, rows_v, zer_v, acc_sh, semg, sems, semi):
    c = lax.axis_index("c")
    s = lax.axis_index("s")
    wid = c * 16 + s
    zero = jnp.zeros((16,), jnp.float32)

    pltpu.async_copy(ei_hbm.at[pl.ds(wid * EPT, EPT)], src_v, semi)
    pltpu.async_copy(ei_hbm.at[pl.ds(E + wid * EPT, EPT)], dst_v, semi)

    def fill_zero(i, _):
        zer_v[i] = zero
        return 0

    lax.fori_loop(0, 128, fill_zero, 0)

    def zero_acc(t, _):
        pltpu.sync_copy(zer_v, acc_sh.at[pl.ds(s * RPT + t * 128, 128)])
        return 0

    lax.fori_loop(0, RPT // 128, zero_acc, 0)
    pltpu.make_async_copy(ei_hbm.at[pl.ds(wid * EPT, EPT)], src_v, semi).wait()
    pltpu.make_async_copy(ei_hbm.at[pl.ds(wid * EPT, EPT)], dst_v, semi).wait()
    plsc.subcore_barrier()

    KH = K // 2

    def fire_g(j, _):
        pltpu.async_copy(feat_hbm.at[src_v.at[pl.ds(j * CH, CH)]],
                         rows_v.at[pl.ds(j * CH, CH)], semg)
        return 0

    def fire_s(j, _):
        pltpu.async_copy(rows_v.at[pl.ds(j * CH, CH)],
                         acc_sh.at[dst_v.at[pl.ds(j * CH, CH)]], sems, add=True)
        return 0

    lax.fori_loop(0, KH, fire_g, 0)
    pltpu.make_async_copy(feat_hbm.at[pl.ds(0, KH * CH)],
                          rows_v.at[pl.ds(0, KH * CH)], semg).wait()
    lax.fori_loop(0, KH, fire_s, 0)
    lax.fori_loop(KH, K, fire_g, 0)
    pltpu.async_copy(feat_hbm.at[src_v.at[pl.ds(K * CH, TAIL)]],
                     rows_v.at[pl.ds(K * CH, TAIL)], semg)
    pltpu.make_async_copy(feat_hbm.at[pl.ds(0, EPT - KH * CH)],
                          rows_v.at[pl.ds(KH * CH, EPT - KH * CH)], semg).wait()
    lax.fori_loop(KH, K, fire_s, 0)
    pltpu.async_copy(rows_v.at[pl.ds(K * CH, TAIL)],
                     acc_sh.at[dst_v.at[pl.ds(K * CH, TAIL)]], sems, add=True)
    pltpu.make_async_copy(rows_v, acc_sh.at[pl.ds(0, EPT)], sems).wait()
    plsc.subcore_barrier()
    sl = pl.ds(s * RPT, RPT)
    pltpu.sync_copy(acc_sh.at[sl], out_hbm.at[c, sl])


def _tc1a_body(x_ref, w1_ref, xw_ref):
    xw_ref[0:N, :] = jnp.dot(x_ref[...], w1_ref[...],
                             preferred_element_type=jnp.float32)
    xw_ref[N:NPAD, :] = jnp.zeros((NPAD - N, 16), jnp.float32)


def _tc1b_body(xw_ref, dinv_ref, xs_ref):
    dcol = dinv_ref[...][:, None]
    xs_ref[...] = xw_ref[...] * dcol


def _tc2_body(accp_ref, xs_ref, dinvw_ref, b1_ref, hs_ref):
    a = accp_ref[0] + accp_ref[1]
    dw = dinvw_ref[...]
    b1w = jnp.concatenate([b1_ref[...]] * 8)
    h = jnp.maximum(dw * (a + xs_ref[...]) + b1w, 0.0)
    hs_ref[...] = h * dw


def _tc3_body(accp2_ref, hs_ref, dinvw_ref, w2big_ref, b2_ref, out_ref):
    m = dinvw_ref[...] * (accp2_ref[0] + accp2_ref[1] + hs_ref[...])
    b2w = jnp.concatenate([b2_ref[...]] * 8)
    out_ref[...] = (jnp.dot(m, w2big_ref[...],
                            preferred_element_type=jnp.float32) + b2w)


def kernel(x, edge_index, W1, b1, W2, b2):
    f32 = jnp.float32
    ei_lin = edge_index.astype(jnp.int32).reshape(2 * E)

    dinv, dinvw = _deg_kernel(ei_lin)

    xw = pl.pallas_call(
        _tc1a_body,
        out_shape=jax.ShapeDtypeStruct((NPAD, 16), f32),
    )(x, W1)
    xs = pl.pallas_call(
        _tc1b_body,
        out_shape=jax.ShapeDtypeStruct((NPAD, 16), f32),
    )(xw, dinv)

    accp = _agg_kernel(ei_lin, xs)

    dinvw_w = dinvw.reshape(NW8, 128)
    hs_w = pl.pallas_call(
        _tc2_body,
        out_shape=jax.ShapeDtypeStruct((NW8, 128), f32),
    )(accp.reshape(2, NW8, 128), xs.reshape(NW8, 128), dinvw_w, b1)

    accp2 = _agg_kernel(ei_lin, hs_w.reshape(NPAD, 16))

    w2big = jnp.kron(jnp.eye(8, dtype=f32), W2.astype(f32))
    out_w = pl.pallas_call(
        _tc3_body,
        out_shape=jax.ShapeDtypeStruct((NW8, 16), f32),
    )(accp2.reshape(2, NW8, 128), hs_w, dinvw_w, w2big, b2)
    return out_w[:N * 2 // 16].reshape(N, 2)


# speedup vs baseline: 61.1596x; 1.0731x over previous
import functools

import jax
import jax.numpy as jnp
from jax import lax
from jax.experimental import pallas as pl
from jax.experimental.pallas import tpu as pltpu
from jax.experimental.pallas import tpu_sc as plsc

N = 10000
NPAD = 10240
NW8 = NPAD // 8
E = 160000
NW = 32
EPT = E // NW
CH = 128
K = EPT // CH
TAIL = EPT - K * CH
EPT_D = E // 16
K_D = EPT_D // CH
TAIL_D = EPT_D - K_D * CH
RPT = NPAD // 16
RPW = NPAD // 32

_mesh = plsc.VectorSubcoreMesh(core_axis_name="c", subcore_axis_name="s")
_sc_params = pltpu.CompilerParams(use_tc_tiling_on_sc=False,
                                  needs_layout_passes=False)


@functools.partial(
    pl.kernel,
    out_type=jax.ShapeDtypeStruct((NPAD, 16), jnp.float32),
    mesh=_mesh,
    compiler_params=_sc_params,
    scratch_types=[
        pltpu.VMEM((EPT_D,), jnp.int32),
        pltpu.VMEM((128,), jnp.float32),
        pltpu.VMEM((RPT,), jnp.float32),
        pltpu.VMEM((RPW,), jnp.float32),
        pltpu.VMEM((RPW, 16), jnp.float32),
        pltpu.VMEM_SHARED((NPAD,), jnp.float32),
        pltpu.SemaphoreType.DMA,
    ],
)
def _deg_kernel(ei_hbm, dinvw_hbm,
                idx_v, ones_v, zer_v, dinv_v, dvw_v, deg_sh, sem):
    c = lax.axis_index("c")
    s = lax.axis_index("s")
    wid = c * 16 + s
    one = jnp.ones((16,), jnp.float32)
    zero = jnp.zeros((16,), jnp.float32)

    def fill_ones(i, _):
        ones_v[pl.ds(i * 16, 16)] = one
        return 0

    lax.fori_loop(0, 8, fill_ones, 0)

    def fill_zero(i, _):
        zer_v[pl.ds(i * 16, 16)] = zero
        return 0

    lax.fori_loop(0, RPT // 16, fill_zero, 0)
    pltpu.sync_copy(ei_hbm.at[pl.ds(E + s * EPT_D, EPT_D)], idx_v)
    pltpu.sync_copy(zer_v, deg_sh.at[pl.ds(s * RPT, RPT)])
    plsc.subcore_barrier()

    def fire(j, _):
        pltpu.async_copy(ones_v.at[pl.ds(0, CH)],
                         deg_sh.at[idx_v.at[pl.ds(j * CH, CH)]], sem, add=True)
        return 0

    lax.fori_loop(0, K_D, fire, 0)
    pltpu.async_copy(ones_v.at[pl.ds(0, TAIL_D)],
                     deg_sh.at[idx_v.at[pl.ds(K_D * CH, TAIL_D)]], sem,
                     add=True)

    def drain(j, _):
        pltpu.make_async_copy(ones_v.at[pl.ds(0, CH)],
                              deg_sh.at[idx_v.at[pl.ds(j * CH, CH)]],
                              sem).wait()
        return 0

    lax.fori_loop(0, K_D, drain, 0)
    pltpu.make_async_copy(ones_v.at[pl.ds(0, TAIL_D)],
                          deg_sh.at[idx_v.at[pl.ds(K_D * CH, TAIL_D)]],
                          sem).wait()
    plsc.subcore_barrier()

    pltpu.sync_copy(deg_sh.at[pl.ds(wid * RPW, RPW)], dinv_v)

    def rsqrt_chunk(i, _):
        d = dinv_v[pl.ds(i * 16, 16)] + 1.0
        bits = plsc.bitcast(d, jnp.int32)
        y = plsc.bitcast(0x5F3759DF - lax.shift_right_logical(bits, 1),
                         jnp.float32)
        half = -0.5 * d
        y = y * (1.5 + half * y * y)
        y = y * (1.5 + half * y * y)
        y = y * (1.5 + half * y * y)
        dinv_v[pl.ds(i * 16, 16)] = y
        return 0

    lax.fori_loop(0, RPW // 16, rsqrt_chunk, 0)

    def splat_row(r, _):
        dvw_v[r] = plsc.load_gather(dinv_v, [jnp.full((16,), r, jnp.int32)])
        return 0

    lax.fori_loop(0, RPW, splat_row, 0)
    pltpu.sync_copy(dvw_v, dinvw_hbm.at[pl.ds(wid * RPW, RPW)])


@functools.partial(
    pl.kernel,
    out_type=jax.ShapeDtypeStruct((2, NPAD, 16), jnp.float32),
    mesh=_mesh,
    compiler_params=_sc_params,
    scratch_types=[
        pltpu.VMEM((EPT,), jnp.int32),
        pltpu.VMEM((EPT,), jnp.int32),
        pltpu.VMEM((EPT, 16), jnp.float32),
        pltpu.VMEM((128, 16), jnp.float32),
        pltpu.VMEM_SHARED((NPAD, 16), jnp.float32),
        pltpu.SemaphoreType.DMA,
        pltpu.SemaphoreType.DMA,
        pltpu.SemaphoreType.DMA,
    ],
)
def _agg_kernel(ei_hbm, feat_hbm, out_hbm,
                src_v, dst_v, rows_v, zer_v, acc_sh, semg, sems, semi):
    c = lax.axis_index("c")
    s = lax.axis_index("s")
    wid = c * 16 + s
    zero = jnp.zeros((16,), jnp.float32)

    pltpu.async_copy(ei_hbm.at[pl.ds(wid * EPT, EPT)], src_v, semi)
    pltpu.async_copy(ei_hbm.at[pl.ds(E + wid * EPT, EPT)], dst_v, semi)

    def fill_zero(i, _):
        zer_v[i] = zero
        return 0

    lax.fori_loop(0, 128, fill_zero, 0)

    def zero_acc(t, _):
        pltpu.sync_copy(zer_v, acc_sh.at[pl.ds(s * RPT + t * 128, 128)])
        return 0

    lax.fori_loop(0, RPT // 128, zero_acc, 0)
    pltpu.make_async_copy(ei_hbm.at[pl.ds(wid * EPT, EPT)], src_v, semi).wait()
    pltpu.make_async_copy(ei_hbm.at[pl.ds(wid * EPT, EPT)], dst_v, semi).wait()
    plsc.subcore_barrier()

    KH = K // 2

    def fire_g(j, _):
        pltpu.async_copy(feat_hbm.at[src_v.at[pl.ds(j * CH, CH)]],
                         rows_v.at[pl.ds(j * CH, CH)], semg)
        return 0

    def fire_s(j, _):
        pltpu.async_copy(rows_v.at[pl.ds(j * CH, CH)],
                         acc_sh.at[dst_v.at[pl.ds(j * CH, CH)]], sems, add=True)
        return 0

    lax.fori_loop(0, KH, fire_g, 0)
    pltpu.make_async_copy(feat_hbm.at[pl.ds(0, KH * CH)],
                          rows_v.at[pl.ds(0, KH * CH)], semg).wait()
    lax.fori_loop(0, KH, fire_s, 0)
    lax.fori_loop(KH, K, fire_g, 0)
    pltpu.async_copy(feat_hbm.at[src_v.at[pl.ds(K * CH, TAIL)]],
                     rows_v.at[pl.ds(K * CH, TAIL)], semg)
    pltpu.make_async_copy(feat_hbm.at[pl.ds(0, EPT - KH * CH)],
                          rows_v.at[pl.ds(KH * CH, EPT - KH * CH)], semg).wait()
    lax.fori_loop(KH, K, fire_s, 0)
    pltpu.async_copy(rows_v.at[pl.ds(K * CH, TAIL)],
                     acc_sh.at[dst_v.at[pl.ds(K * CH, TAIL)]], sems, add=True)
    pltpu.make_async_copy(rows_v, acc_sh.at[pl.ds(0, EPT)], sems).wait()
    plsc.subcore_barrier()
    sl = pl.ds(s * RPT, RPT)
    pltpu.sync_copy(acc_sh.at[sl], out_hbm.at[c, sl])


def _tc1a_body(x_ref, w1_ref, xw_ref):
    xw_ref[0:N, :] = jnp.dot(x_ref[...], w1_ref[...],
                             preferred_element_type=jnp.float32)
    xw_ref[N:NPAD, :] = jnp.zeros((NPAD - N, 16), jnp.float32)


def _tc1b_body(xw_ref, dinvw_ref, xs_ref):
    xs_ref[...] = xw_ref[...] * dinvw_ref[...]


def _tc2_body(accp_ref, xs_ref, dinvw_ref, b1_ref, hs_ref):
    a = accp_ref[0] + accp_ref[1]
    dw = dinvw_ref[...]
    b1w = jnp.concatenate([b1_ref[...]] * 8)
    h = jnp.maximum(dw * (a + xs_ref[...]) + b1w, 0.0)
    hs_ref[...] = h * dw


def _tc3_body(accp2_ref, hs_ref, dinvw_ref, w2big_ref, b2_ref, out_ref):
    m = dinvw_ref[...] * (accp2_ref[0] + accp2_ref[1] + hs_ref[...])
    b2w = jnp.concatenate([b2_ref[...]] * 8)
    out_ref[...] = jnp.dot(m, w2big_ref[...],
                           preferred_element_type=jnp.float32) + b2w


def kernel(x, edge_index, W1, b1, W2, b2):
    f32 = jnp.float32
    ei_lin = edge_index.astype(jnp.int32).reshape(2 * E)

    dinvw = _deg_kernel(ei_lin)

    dinvw_w = dinvw.reshape(NW8, 128)
    xw = pl.pallas_call(
        _tc1a_body,
        out_shape=jax.ShapeDtypeStruct((NPAD, 16), f32),
    )(x, W1)
    xs_w = pl.pallas_call(
        _tc1b_body,
        out_shape=jax.ShapeDtypeStruct((NW8, 128), f32),
    )(xw.reshape(NW8, 128), dinvw_w)
    xs = xs_w.reshape(NPAD, 16)

    accp = _agg_kernel(ei_lin, xs)

    hs_w = pl.pallas_call(
        _tc2_body,
        out_shape=jax.ShapeDtypeStruct((NW8, 128), f32),
    )(accp.reshape(2, NW8, 128), xs_w, dinvw_w, b1)

    accp2 = _agg_kernel(ei_lin, hs_w.reshape(NPAD, 16))

    w2big = jnp.kron(jnp.eye(8, dtype=f32), W2.astype(f32))
    out_w = pl.pallas_call(
        _tc3_body,
        out_shape=jax.ShapeDtypeStruct((NW8, 16), f32),
    )(accp2.reshape(2, NW8, 128), hs_w, dinvw_w, w2big, b2)
    return out_w[:N * 2 // 16].reshape(N, 2)
